# Initial kernel scaffold; baseline (speedup 1.0000x reference)
import jax
import jax.numpy as jnp
from jax.experimental import pallas as pl


def kernel(x, edge_index, W1, b1, W2, b2, Wfc, bfc):
    raise NotImplementedError("write your pallas kernel here")


# speedup vs baseline: 33.5786x; 33.5786x over previous
import jax
import jax.numpy as jnp
from jax import lax
from jax.experimental import pallas as pl
from jax.experimental.pallas import tpu as pltpu
from jax.experimental.pallas import tpu_sc as plsc

N = 10000
E = 320000
F_IN = 128
HID = 16
NC = 2
NS = 16
NW = NC * NS
L = 16
CHUNK = 128
CPT = 79
E_PAD = NW * CPT * CHUNK
EPT = CPT * CHUNK
N_PAD = 10112
ZROWS = N_PAD // NS
ZK = 8
ZC = ZROWS // ZK

_MESH = plsc.VectorSubcoreMesh(core_axis_name="c", subcore_axis_name="s")
_SC_PARAMS = pltpu.CompilerParams(use_tc_tiling_on_sc=False)


def _worker_id():
    return lax.axis_index("c") * NS + lax.axis_index("s")


def _fill_rows(ref, nrows, vec):
    def fill(i, _):
        ref[i, :] = vec
        return 0

    lax.fori_loop(0, nrows, fill, 0)


def _zero_acc(idx_v, zb, acc):
    for k in range(ZK):
        pltpu.sync_copy(zb, acc.at[idx_v.at[k]])


def _read_acc(idx_v, rb, acc):
    for k in range(ZK):
        pltpu.sync_copy(acc.at[idx_v.at[k]], rb.at[k])


def _deg_body(dst_hbm, ridx_hbm, out_hbm, dst_v, idx_v, ones_v, zb, rb, acc):
    sid = lax.axis_index("s")
    wid = _worker_id()
    pltpu.sync_copy(dst_hbm.at[wid], dst_v)
    pltpu.sync_copy(ridx_hbm.at[sid], idx_v)
    _fill_rows(ones_v, CHUNK, jnp.ones((L,), jnp.float32))
    _fill_rows(zb, ZC, jnp.zeros((L,), jnp.float32))
    _zero_acc(idx_v, zb, acc)
    plsc.subcore_barrier()

    def body(j, _):
        pltpu.sync_copy(ones_v, acc.at[dst_v.at[j]], add=True)
        return 0

    lax.fori_loop(0, CPT, body, 0)
    plsc.subcore_barrier()
    _read_acc(idx_v, rb, acc)
    pltpu.sync_copy(rb, out_hbm.at[wid])


_deg_kernel = pl.kernel(
    _deg_body,
    out_type=jax.ShapeDtypeStruct((NW, ZK, ZC, HID), jnp.float32),
    mesh=_MESH,
    scratch_types=[
        pltpu.VMEM((CPT, CHUNK), jnp.int32),
        pltpu.VMEM((ZK, ZC), jnp.int32),
        pltpu.VMEM((CHUNK, HID), jnp.float32),
        pltpu.VMEM((ZC, HID), jnp.float32),
        pltpu.VMEM((ZK, ZC, HID), jnp.float32),
        pltpu.VMEM_SHARED((N_PAD, HID), jnp.float32),
    ],
    compiler_params=_SC_PARAMS,
)


def _msg_body(y_hbm, src_hbm, dst_hbm, ridx_hbm, out_hbm, src_v, dst_v,
              idx_v, rows, zb, rb, acc, sem):
    sid = lax.axis_index("s")
    wid = _worker_id()
    pltpu.sync_copy(src_hbm.at[wid], src_v)
    pltpu.sync_copy(dst_hbm.at[wid], dst_v)
    pltpu.sync_copy(ridx_hbm.at[sid], idx_v)
    _fill_rows(zb, ZC, jnp.zeros((L,), jnp.float32))
    _zero_acc(idx_v, zb, acc)
    plsc.subcore_barrier()

    def body(j, _):
        pltpu.async_copy(y_hbm.at[src_v.at[j]], rows, sem).wait()
        pltpu.sync_copy(rows, acc.at[dst_v.at[j]], add=True)
        return 0

    lax.fori_loop(0, CPT, body, 0)
    plsc.subcore_barrier()
    _read_acc(idx_v, rb, acc)
    pltpu.sync_copy(rb, out_hbm.at[wid])


_msg_kernel = pl.kernel(
    _msg_body,
    out_type=jax.ShapeDtypeStruct((NW, ZK, ZC, HID), jnp.float32),
    mesh=_MESH,
    scratch_types=[
        pltpu.VMEM((CPT, CHUNK), jnp.int32),
        pltpu.VMEM((CPT, CHUNK), jnp.int32),
        pltpu.VMEM((ZK, ZC), jnp.int32),
        pltpu.VMEM((CHUNK, HID), jnp.float32),
        pltpu.VMEM((ZC, HID), jnp.float32),
        pltpu.VMEM((ZK, ZC, HID), jnp.float32),
        pltpu.VMEM_SHARED((N_PAD, HID), jnp.float32),
        pltpu.SemaphoreType.DMA,
    ],
    compiler_params=_SC_PARAMS,
)


def _edge_body(a_hbm, c_hbm, src_hbm, dst_hbm, out_hbm, a_v, c_v, src_v,
               dst_v, o_v):
    wid = _worker_id()
    pltpu.sync_copy(a_hbm, a_v)
    pltpu.sync_copy(c_hbm, c_v)
    pltpu.sync_copy(src_hbm.at[wid], src_v)
    pltpu.sync_copy(dst_hbm.at[wid], dst_v)

    def body(j, _):
        for k in range(CHUNK // L):
            sv = src_v[j, pl.ds(k * L, L)]
            dv = dst_v[j, pl.ds(k * L, L)]
            av = plsc.load_gather(a_v, [sv])
            cv = plsc.load_gather(c_v, [dv])
            o_v[pl.ds(j * CHUNK + k * L, L)] = av + cv
        return 0

    lax.fori_loop(0, CPT, body, 0)
    pltpu.sync_copy(o_v, out_hbm.at[pl.ds(wid * EPT, EPT)])


_edge_kernel = pl.kernel(
    _edge_body,
    out_type=jax.ShapeDtypeStruct((E_PAD,), jnp.float32),
    mesh=_MESH,
    scratch_types=[
        pltpu.VMEM((N_PAD,), jnp.float32),
        pltpu.VMEM((N_PAD,), jnp.float32),
        pltpu.VMEM((CPT, CHUNK), jnp.int32),
        pltpu.VMEM((CPT, CHUNK), jnp.int32),
        pltpu.VMEM((EPT,), jnp.float32),
    ],
    compiler_params=pltpu.CompilerParams(needs_layout_passes=False),
)


def _tc1_body(x_ref, w1_ref, degp_ref, y1_ref, dinv_ref):
    deg = (degp_ref[0:N, 0:1] + degp_ref[N_PAD:N_PAD + N, 0:1]) + 1.0
    dinv = lax.rsqrt(deg)
    y1 = dinv * jnp.dot(x_ref[...], w1_ref[...],
                        preferred_element_type=jnp.float32)
    y1_ref[...] = y1
    dinv_ref[...] = dinv


_tc1 = pl.pallas_call(
    _tc1_body,
    out_shape=(
        jax.ShapeDtypeStruct((N, HID), jnp.float32),
        jax.ShapeDtypeStruct((N, 1), jnp.float32),
    ),
)


def _tc2_body(sp_ref, y1_ref, dinv_ref, w2_ref, b1_ref, y2_ref):
    s1 = sp_ref[0:N, :] + sp_ref[N_PAD:N_PAD + N, :]
    dinv = dinv_ref[...]
    h1 = jnp.maximum(dinv * (s1 + y1_ref[...]) + b1_ref[...], 0.0)
    y2_ref[...] = dinv * jnp.dot(h1, w2_ref[...],
                                 preferred_element_type=jnp.float32)


_tc2 = pl.pallas_call(
    _tc2_body,
    out_shape=jax.ShapeDtypeStruct((N, HID), jnp.float32),
)


def _tc3_body(sp_ref, y2_ref, dinv_ref, wfc_ref, b2_ref, bfc_ref, a_ref,
              c_ref):
    s2 = sp_ref[0:N, :] + sp_ref[N_PAD:N_PAD + N, :]
    dinv = dinv_ref[...]
    h2 = jnp.maximum(dinv * (s2 + y2_ref[...]) + b2_ref[...], 0.0)
    a = jnp.dot(h2, wfc_ref[0:HID, :],
                preferred_element_type=jnp.float32) + bfc_ref[...]
    c = jnp.dot(h2, wfc_ref[HID:2 * HID, :],
                preferred_element_type=jnp.float32)
    a_ref[0:N, :] = a
    a_ref[N:N_PAD, :] = jnp.zeros((N_PAD - N, 1), jnp.float32)
    c_ref[0:N, :] = c
    c_ref[N:N_PAD, :] = jnp.zeros((N_PAD - N, 1), jnp.float32)


_tc3 = pl.pallas_call(
    _tc3_body,
    out_shape=(
        jax.ShapeDtypeStruct((N_PAD, 1), jnp.float32),
        jax.ShapeDtypeStruct((N_PAD, 1), jnp.float32),
    ),
)


@jax.jit
def kernel(x, edge_index, W1, b1, W2, b2, Wfc, bfc):
    src = edge_index[0].astype(jnp.int32)
    dst = edge_index[1].astype(jnp.int32)
    pad = E_PAD - E
    src_p = jnp.concatenate([src, jnp.zeros((pad,), jnp.int32)])
    dst_p = jnp.concatenate([dst, jnp.full((pad,), N, jnp.int32)])
    src_p = src_p.reshape(NW, CPT, CHUNK)
    dst_p = dst_p.reshape(NW, CPT, CHUNK)
    ridx = jnp.arange(N_PAD, dtype=jnp.int32).reshape(NS, ZK, ZC)

    degp = _deg_kernel(dst_p, ridx).reshape(NC * N_PAD, HID)
    y1, dinv = _tc1(x, W1, degp)
    sp1 = _msg_kernel(y1, src_p, dst_p, ridx).reshape(NC * N_PAD, HID)
    y2 = _tc2(sp1, y1, dinv, W2, b1.reshape(1, HID))
    sp2 = _msg_kernel(y2, src_p, dst_p, ridx).reshape(NC * N_PAD, HID)
    a_pad, c_pad = _tc3(sp2, y2, dinv, Wfc, b2.reshape(1, HID),
                        bfc.reshape(1, 1))
    pred = _edge_kernel(a_pad.reshape(N_PAD), c_pad.reshape(N_PAD), src_p,
                        dst_p)
    return pred[:E]


# speedup vs baseline: 38.9010x; 1.1585x over previous
import jax
import jax.numpy as jnp
from jax import lax
from jax.experimental import pallas as pl
from jax.experimental.pallas import tpu as pltpu
from jax.experimental.pallas import tpu_sc as plsc

N = 10000
E = 320000
F_IN = 128
HID = 16
NC = 2
NS = 16
NW = NC * NS
L = 16
CHUNK = 128
CPT = 80
NBUF = 4
DGRP = 8
E_PAD = NW * CPT * CHUNK
EPT = CPT * CHUNK
N_PAD = 10112
ZROWS = N_PAD // NS
ZK = 8
ZC = ZROWS // ZK

_MESH = plsc.VectorSubcoreMesh(core_axis_name="c", subcore_axis_name="s")
_SC_PARAMS = pltpu.CompilerParams(use_tc_tiling_on_sc=False)


def _worker_id():
    return lax.axis_index("c") * NS + lax.axis_index("s")


def _fill_rows(ref, nrows, vec):
    def fill(i, _):
        ref[i, :] = vec
        return 0

    lax.fori_loop(0, nrows, fill, 0)


def _zero_acc(idx_v, zb, acc):
    for k in range(ZK):
        pltpu.sync_copy(zb, acc.at[idx_v.at[k]])


def _read_acc(idx_v, rb, acc):
    for k in range(ZK):
        pltpu.sync_copy(acc.at[idx_v.at[k]], rb.at[k])


def _deg_body(dst_hbm, ridx_hbm, out_hbm, dst_v, idx_v, ones_v, zb, rb, acc,
              sem):
    sid = lax.axis_index("s")
    wid = _worker_id()
    pltpu.sync_copy(dst_hbm.at[wid], dst_v)
    pltpu.sync_copy(ridx_hbm.at[sid], idx_v)
    _fill_rows(ones_v, CHUNK, jnp.ones((L,), jnp.float32))
    _fill_rows(zb, ZC, jnp.zeros((L,), jnp.float32))
    _zero_acc(idx_v, zb, acc)
    plsc.subcore_barrier()

    def body(t, _):
        for b in range(DGRP):
            pltpu.async_copy(ones_v, acc.at[dst_v.at[t * DGRP + b]], sem,
                             add=True)
        for b in range(DGRP):
            pltpu.make_async_copy(ones_v, acc.at[dst_v.at[t * DGRP + b]],
                                  sem).wait()
        return 0

    lax.fori_loop(0, CPT // DGRP, body, 0)
    plsc.subcore_barrier()
    _read_acc(idx_v, rb, acc)
    pltpu.sync_copy(rb, out_hbm.at[wid])


_deg_kernel = pl.kernel(
    _deg_body,
    out_type=jax.ShapeDtypeStruct((NW, ZK, ZC, HID), jnp.float32),
    mesh=_MESH,
    scratch_types=[
        pltpu.VMEM((CPT, CHUNK), jnp.int32),
        pltpu.VMEM((ZK, ZC), jnp.int32),
        pltpu.VMEM((CHUNK, HID), jnp.float32),
        pltpu.VMEM((ZC, HID), jnp.float32),
        pltpu.VMEM((ZK, ZC, HID), jnp.float32),
        pltpu.VMEM_SHARED((N_PAD, HID), jnp.float32),
        pltpu.SemaphoreType.DMA,
    ],
    compiler_params=_SC_PARAMS,
)


def _msg_body(y_hbm, src_hbm, dst_hbm, ridx_hbm, out_hbm, src_v, dst_v,
              idx_v, rows, zb, rb, acc, *sems):
    sid = lax.axis_index("s")
    wid = _worker_id()
    pltpu.sync_copy(src_hbm.at[wid], src_v)
    pltpu.sync_copy(dst_hbm.at[wid], dst_v)
    pltpu.sync_copy(ridx_hbm.at[sid], idx_v)
    _fill_rows(zb, ZC, jnp.zeros((L,), jnp.float32))
    _zero_acc(idx_v, zb, acc)
    plsc.subcore_barrier()

    for b in range(NBUF):
        pltpu.async_copy(y_hbm.at[src_v.at[b]], rows.at[b], sems[b])

    def body(t, _):
        for b in range(NBUF):
            j = t * NBUF + b
            pltpu.make_async_copy(y_hbm.at[pl.ds(0, CHUNK)], rows.at[b],
                                  sems[b]).wait()
            pltpu.sync_copy(rows.at[b], acc.at[dst_v.at[j]], add=True)
            pltpu.async_copy(y_hbm.at[src_v.at[j + NBUF]], rows.at[b],
                             sems[b])
        return 0

    lax.fori_loop(0, CPT // NBUF - 1, body, 0)
    for b in range(NBUF):
        j = CPT - NBUF + b
        pltpu.make_async_copy(y_hbm.at[pl.ds(0, CHUNK)], rows.at[b],
                              sems[b]).wait()
        pltpu.sync_copy(rows.at[b], acc.at[dst_v.at[j]], add=True)
    plsc.subcore_barrier()
    _read_acc(idx_v, rb, acc)
    pltpu.sync_copy(rb, out_hbm.at[wid])


_msg_kernel = pl.kernel(
    _msg_body,
    out_type=jax.ShapeDtypeStruct((NW, ZK, ZC, HID), jnp.float32),
    mesh=_MESH,
    scratch_types=[
        pltpu.VMEM((CPT, CHUNK), jnp.int32),
        pltpu.VMEM((CPT, CHUNK), jnp.int32),
        pltpu.VMEM((ZK, ZC), jnp.int32),
        pltpu.VMEM((NBUF, CHUNK, HID), jnp.float32),
        pltpu.VMEM((ZC, HID), jnp.float32),
        pltpu.VMEM((ZK, ZC, HID), jnp.float32),
        pltpu.VMEM_SHARED((N_PAD, HID), jnp.float32),
    ] + [pltpu.SemaphoreType.DMA] * NBUF,
    compiler_params=_SC_PARAMS,
)


def _edge_body(a_hbm, c_hbm, src_hbm, dst_hbm, out_hbm, a_v, c_v, src_v,
               dst_v, o_v):
    wid = _worker_id()
    pltpu.sync_copy(a_hbm, a_v)
    pltpu.sync_copy(c_hbm, c_v)
    pltpu.sync_copy(src_hbm.at[wid], src_v)
    pltpu.sync_copy(dst_hbm.at[wid], dst_v)

    def body(j, _):
        for k in range(CHUNK // L):
            sv = src_v[j, pl.ds(k * L, L)]
            dv = dst_v[j, pl.ds(k * L, L)]
            av = plsc.load_gather(a_v, [sv])
            cv = plsc.load_gather(c_v, [dv])
            o_v[pl.ds(j * CHUNK + k * L, L)] = av + cv
        return 0

    lax.fori_loop(0, CPT, body, 0)
    pltpu.sync_copy(o_v, out_hbm.at[pl.ds(wid * EPT, EPT)])


_edge_kernel = pl.kernel(
    _edge_body,
    out_type=jax.ShapeDtypeStruct((E_PAD,), jnp.float32),
    mesh=_MESH,
    scratch_types=[
        pltpu.VMEM((N_PAD,), jnp.float32),
        pltpu.VMEM((N_PAD,), jnp.float32),
        pltpu.VMEM((CPT, CHUNK), jnp.int32),
        pltpu.VMEM((CPT, CHUNK), jnp.int32),
        pltpu.VMEM((EPT,), jnp.float32),
    ],
    compiler_params=pltpu.CompilerParams(needs_layout_passes=False),
)


def _tc1_body(x_ref, w1_ref, degp_ref, y1_ref, dinv_ref):
    deg = (degp_ref[0:N, 0:1] + degp_ref[N_PAD:N_PAD + N, 0:1]) + 1.0
    dinv = lax.rsqrt(deg)
    y1 = dinv * jnp.dot(x_ref[...], w1_ref[...],
                        preferred_element_type=jnp.float32)
    y1_ref[...] = y1
    dinv_ref[...] = dinv


_tc1 = pl.pallas_call(
    _tc1_body,
    out_shape=(
        jax.ShapeDtypeStruct((N, HID), jnp.float32),
        jax.ShapeDtypeStruct((N, 1), jnp.float32),
    ),
)


def _tc2_body(sp_ref, y1_ref, dinv_ref, w2_ref, b1_ref, y2_ref):
    s1 = sp_ref[0:N, :] + sp_ref[N_PAD:N_PAD + N, :]
    dinv = dinv_ref[...]
    h1 = jnp.maximum(dinv * (s1 + y1_ref[...]) + b1_ref[...], 0.0)
    y2_ref[...] = dinv * jnp.dot(h1, w2_ref[...],
                                 preferred_element_type=jnp.float32)


_tc2 = pl.pallas_call(
    _tc2_body,
    out_shape=jax.ShapeDtypeStruct((N, HID), jnp.float32),
)


def _tc3_body(sp_ref, y2_ref, dinv_ref, wfc_ref, b2_ref, bfc_ref, a_ref,
              c_ref):
    s2 = sp_ref[0:N, :] + sp_ref[N_PAD:N_PAD + N, :]
    dinv = dinv_ref[...]
    h2 = jnp.maximum(dinv * (s2 + y2_ref[...]) + b2_ref[...], 0.0)
    a = jnp.dot(h2, wfc_ref[0:HID, :],
                preferred_element_type=jnp.float32) + bfc_ref[...]
    c = jnp.dot(h2, wfc_ref[HID:2 * HID, :],
                preferred_element_type=jnp.float32)
    a_ref[0:N, :] = a
    a_ref[N:N_PAD, :] = jnp.zeros((N_PAD - N, 1), jnp.float32)
    c_ref[0:N, :] = c
    c_ref[N:N_PAD, :] = jnp.zeros((N_PAD - N, 1), jnp.float32)


_tc3 = pl.pallas_call(
    _tc3_body,
    out_shape=(
        jax.ShapeDtypeStruct((N_PAD, 1), jnp.float32),
        jax.ShapeDtypeStruct((N_PAD, 1), jnp.float32),
    ),
)


@jax.jit
def kernel(x, edge_index, W1, b1, W2, b2, Wfc, bfc):
    src = edge_index[0].astype(jnp.int32)
    dst = edge_index[1].astype(jnp.int32)
    pad = E_PAD - E
    src_p = jnp.concatenate([src, jnp.zeros((pad,), jnp.int32)])
    dst_p = jnp.concatenate([dst, jnp.full((pad,), N, jnp.int32)])
    src_p = src_p.reshape(NW, CPT, CHUNK)
    dst_p = dst_p.reshape(NW, CPT, CHUNK)
    ridx = jnp.arange(N_PAD, dtype=jnp.int32).reshape(NS, ZK, ZC)

    degp = _deg_kernel(dst_p, ridx).reshape(NC * N_PAD, HID)
    y1, dinv = _tc1(x, W1, degp)
    sp1 = _msg_kernel(y1, src_p, dst_p, ridx).reshape(NC * N_PAD, HID)
    y2 = _tc2(sp1, y1, dinv, W2, b1.reshape(1, HID))
    sp2 = _msg_kernel(y2, src_p, dst_p, ridx).reshape(NC * N_PAD, HID)
    a_pad, c_pad = _tc3(sp2, y2, dinv, Wfc, b2.reshape(1, HID),
                        bfc.reshape(1, 1))
    pred = _edge_kernel(a_pad.reshape(N_PAD), c_pad.reshape(N_PAD), src_p,
                        dst_p)
    return pred[:E]


# speedup vs baseline: 38.9448x; 1.0011x over previous
import jax
import jax.numpy as jnp
from jax import lax
from jax.experimental import pallas as pl
from jax.experimental.pallas import tpu as pltpu
from jax.experimental.pallas import tpu_sc as plsc

N = 10000
E = 320000
F_IN = 128
HID = 16
NC = 2
NS = 16
NW = NC * NS
L = 16
CHUNK = 128
CPT = 80
NBUF = 8
DGRP = 8
E_PAD = NW * CPT * CHUNK
EPT = CPT * CHUNK
N_PAD = 10112
ZROWS = N_PAD // NS
ZK = 8
ZC = ZROWS // ZK

_MESH = plsc.VectorSubcoreMesh(core_axis_name="c", subcore_axis_name="s")
_SC_PARAMS = pltpu.CompilerParams(use_tc_tiling_on_sc=False)


def _worker_id():
    return lax.axis_index("c") * NS + lax.axis_index("s")


def _fill_rows(ref, nrows, vec):
    def fill(i, _):
        ref[i, :] = vec
        return 0

    lax.fori_loop(0, nrows, fill, 0)


def _zero_acc(idx_v, zb, acc):
    for k in range(ZK):
        pltpu.sync_copy(zb, acc.at[idx_v.at[k]])


def _read_acc(idx_v, rb, acc):
    for k in range(ZK):
        pltpu.sync_copy(acc.at[idx_v.at[k]], rb.at[k])


def _deg_body(dst_hbm, ridx_hbm, out_hbm, dst_v, idx_v, ones_v, zb, rb, acc,
              sem):
    sid = lax.axis_index("s")
    wid = _worker_id()
    pltpu.sync_copy(dst_hbm.at[wid], dst_v)
    pltpu.sync_copy(ridx_hbm.at[sid], idx_v)
    _fill_rows(ones_v, CHUNK, jnp.ones((L,), jnp.float32))
    _fill_rows(zb, ZC, jnp.zeros((L,), jnp.float32))
    _zero_acc(idx_v, zb, acc)
    plsc.subcore_barrier()

    def body(t, _):
        for b in range(DGRP):
            pltpu.async_copy(ones_v, acc.at[dst_v.at[t * DGRP + b]], sem,
                             add=True)
        for b in range(DGRP):
            pltpu.make_async_copy(ones_v, acc.at[dst_v.at[t * DGRP + b]],
                                  sem).wait()
        return 0

    lax.fori_loop(0, CPT // DGRP, body, 0)
    plsc.subcore_barrier()
    _read_acc(idx_v, rb, acc)
    pltpu.sync_copy(rb, out_hbm.at[wid])


_deg_kernel = pl.kernel(
    _deg_body,
    out_type=jax.ShapeDtypeStruct((NW, ZK, ZC, HID), jnp.float32),
    mesh=_MESH,
    scratch_types=[
        pltpu.VMEM((CPT, CHUNK), jnp.int32),
        pltpu.VMEM((ZK, ZC), jnp.int32),
        pltpu.VMEM((CHUNK, HID), jnp.float32),
        pltpu.VMEM((ZC, HID), jnp.float32),
        pltpu.VMEM((ZK, ZC, HID), jnp.float32),
        pltpu.VMEM_SHARED((N_PAD, HID), jnp.float32),
        pltpu.SemaphoreType.DMA,
    ],
    compiler_params=_SC_PARAMS,
)


def _msg_body(y_hbm, src_hbm, dst_hbm, ridx_hbm, out_hbm, src_v, dst_v,
              idx_v, rows, zb, rb, acc, *sems):
    sid = lax.axis_index("s")
    wid = _worker_id()
    pltpu.sync_copy(src_hbm.at[wid], src_v)
    pltpu.sync_copy(dst_hbm.at[wid], dst_v)
    pltpu.sync_copy(ridx_hbm.at[sid], idx_v)
    _fill_rows(zb, ZC, jnp.zeros((L,), jnp.float32))
    _zero_acc(idx_v, zb, acc)
    plsc.subcore_barrier()

    for b in range(NBUF):
        pltpu.async_copy(y_hbm.at[src_v.at[b]], rows.at[b], sems[b])

    def body(t, _):
        for b in range(NBUF):
            j = t * NBUF + b
            pltpu.make_async_copy(y_hbm.at[pl.ds(0, CHUNK)], rows.at[b],
                                  sems[b]).wait()
            pltpu.sync_copy(rows.at[b], acc.at[dst_v.at[j]], add=True)
            pltpu.async_copy(y_hbm.at[src_v.at[j + NBUF]], rows.at[b],
                             sems[b])
        return 0

    lax.fori_loop(0, CPT // NBUF - 1, body, 0)
    for b in range(NBUF):
        j = CPT - NBUF + b
        pltpu.make_async_copy(y_hbm.at[pl.ds(0, CHUNK)], rows.at[b],
                              sems[b]).wait()
        pltpu.sync_copy(rows.at[b], acc.at[dst_v.at[j]], add=True)
    plsc.subcore_barrier()
    _read_acc(idx_v, rb, acc)
    pltpu.sync_copy(rb, out_hbm.at[wid])


_msg_kernel = pl.kernel(
    _msg_body,
    out_type=jax.ShapeDtypeStruct((NW, ZK, ZC, HID), jnp.float32),
    mesh=_MESH,
    scratch_types=[
        pltpu.VMEM((CPT, CHUNK), jnp.int32),
        pltpu.VMEM((CPT, CHUNK), jnp.int32),
        pltpu.VMEM((ZK, ZC), jnp.int32),
        pltpu.VMEM((NBUF, CHUNK, HID), jnp.float32),
        pltpu.VMEM((ZC, HID), jnp.float32),
        pltpu.VMEM((ZK, ZC, HID), jnp.float32),
        pltpu.VMEM_SHARED((N_PAD, HID), jnp.float32),
    ] + [pltpu.SemaphoreType.DMA] * NBUF,
    compiler_params=_SC_PARAMS,
)


def _edge_body(a_hbm, c_hbm, src_hbm, dst_hbm, out_hbm, a_v, c_v, src_v,
               dst_v, o_v):
    wid = _worker_id()
    pltpu.sync_copy(a_hbm, a_v)
    pltpu.sync_copy(c_hbm, c_v)
    pltpu.sync_copy(src_hbm.at[wid], src_v)
    pltpu.sync_copy(dst_hbm.at[wid], dst_v)

    def body(j, _):
        for k in range(CHUNK // L):
            sv = src_v[j, pl.ds(k * L, L)]
            dv = dst_v[j, pl.ds(k * L, L)]
            av = plsc.load_gather(a_v, [sv])
            cv = plsc.load_gather(c_v, [dv])
            o_v[pl.ds(j * CHUNK + k * L, L)] = av + cv
        return 0

    lax.fori_loop(0, CPT, body, 0)
    pltpu.sync_copy(o_v, out_hbm.at[pl.ds(wid * EPT, EPT)])


_edge_kernel = pl.kernel(
    _edge_body,
    out_type=jax.ShapeDtypeStruct((E_PAD,), jnp.float32),
    mesh=_MESH,
    scratch_types=[
        pltpu.VMEM((N_PAD,), jnp.float32),
        pltpu.VMEM((N_PAD,), jnp.float32),
        pltpu.VMEM((CPT, CHUNK), jnp.int32),
        pltpu.VMEM((CPT, CHUNK), jnp.int32),
        pltpu.VMEM((EPT,), jnp.float32),
    ],
    compiler_params=pltpu.CompilerParams(needs_layout_passes=False),
)


def _tc1_body(x_ref, w1_ref, degp_ref, y1_ref, dinv_ref):
    deg = (degp_ref[0:N, 0:1] + degp_ref[N_PAD:N_PAD + N, 0:1]) + 1.0
    dinv = lax.rsqrt(deg)
    y1 = dinv * jnp.dot(x_ref[...], w1_ref[...],
                        preferred_element_type=jnp.float32)
    y1_ref[...] = y1
    dinv_ref[...] = dinv


_tc1 = pl.pallas_call(
    _tc1_body,
    out_shape=(
        jax.ShapeDtypeStruct((N, HID), jnp.float32),
        jax.ShapeDtypeStruct((N, 1), jnp.float32),
    ),
)


def _tc2_body(sp_ref, y1_ref, dinv_ref, w2_ref, b1_ref, y2_ref):
    s1 = sp_ref[0:N, :] + sp_ref[N_PAD:N_PAD + N, :]
    dinv = dinv_ref[...]
    h1 = jnp.maximum(dinv * (s1 + y1_ref[...]) + b1_ref[...], 0.0)
    y2_ref[...] = dinv * jnp.dot(h1, w2_ref[...],
                                 preferred_element_type=jnp.float32)


_tc2 = pl.pallas_call(
    _tc2_body,
    out_shape=jax.ShapeDtypeStruct((N, HID), jnp.float32),
)


def _tc3_body(sp_ref, y2_ref, dinv_ref, wfc_ref, b2_ref, bfc_ref, a_ref,
              c_ref):
    s2 = sp_ref[0:N, :] + sp_ref[N_PAD:N_PAD + N, :]
    dinv = dinv_ref[...]
    h2 = jnp.maximum(dinv * (s2 + y2_ref[...]) + b2_ref[...], 0.0)
    a = jnp.dot(h2, wfc_ref[0:HID, :],
                preferred_element_type=jnp.float32) + bfc_ref[...]
    c = jnp.dot(h2, wfc_ref[HID:2 * HID, :],
                preferred_element_type=jnp.float32)
    a_ref[0:N, :] = a
    a_ref[N:N_PAD, :] = jnp.zeros((N_PAD - N, 1), jnp.float32)
    c_ref[0:N, :] = c
    c_ref[N:N_PAD, :] = jnp.zeros((N_PAD - N, 1), jnp.float32)


_tc3 = pl.pallas_call(
    _tc3_body,
    out_shape=(
        jax.ShapeDtypeStruct((N_PAD, 1), jnp.float32),
        jax.ShapeDtypeStruct((N_PAD, 1), jnp.float32),
    ),
)


@jax.jit
def kernel(x, edge_index, W1, b1, W2, b2, Wfc, bfc):
    src = edge_index[0].astype(jnp.int32)
    dst = edge_index[1].astype(jnp.int32)
    pad = E_PAD - E
    src_p = jnp.concatenate([src, jnp.zeros((pad,), jnp.int32)])
    dst_p = jnp.concatenate([dst, jnp.full((pad,), N, jnp.int32)])
    src_p = src_p.reshape(NW, CPT, CHUNK)
    dst_p = dst_p.reshape(NW, CPT, CHUNK)
    ridx = jnp.arange(N_PAD, dtype=jnp.int32).reshape(NS, ZK, ZC)

    degp = _deg_kernel(dst_p, ridx).reshape(NC * N_PAD, HID)
    y1, dinv = _tc1(x, W1, degp)
    sp1 = _msg_kernel(y1, src_p, dst_p, ridx).reshape(NC * N_PAD, HID)
    y2 = _tc2(sp1, y1, dinv, W2, b1.reshape(1, HID))
    sp2 = _msg_kernel(y2, src_p, dst_p, ridx).reshape(NC * N_PAD, HID)
    a_pad, c_pad = _tc3(sp2, y2, dinv, Wfc, b2.reshape(1, HID),
                        bfc.reshape(1, 1))
    pred = _edge_kernel(a_pad.reshape(N_PAD), c_pad.reshape(N_PAD), src_p,
                        dst_p)
    return pred[:E]


# speedup vs baseline: 53.9405x; 1.3850x over previous
import jax
import jax.numpy as jnp
from jax import lax
from jax.experimental import pallas as pl
from jax.experimental.pallas import tpu as pltpu
from jax.experimental.pallas import tpu_sc as plsc

N = 10000
E = 320000
F_IN = 128
HID = 16
NC = 2
NS = 16
NW = NC * NS
L = 16
CHUNK = 128
CPT = 80
NBUF = 8
DGRP = 8
E_PAD = NW * CPT * CHUNK
EPT = CPT * CHUNK
N_PAD = 10112
ZROWS = N_PAD // NS
ZK = 8
ZC = ZROWS // ZK

_MESH = plsc.VectorSubcoreMesh(core_axis_name="c", subcore_axis_name="s")
_SC_PARAMS = pltpu.CompilerParams(use_tc_tiling_on_sc=False)


def _worker_id():
    return lax.axis_index("c") * NS + lax.axis_index("s")


def _fill_rows(ref, nrows, vec):
    def fill(i, _):
        ref[i, :] = vec
        return 0

    lax.fori_loop(0, nrows, fill, 0)


def _zero_acc(idx_v, zb, acc):
    for k in range(ZK):
        pltpu.sync_copy(zb, acc.at[idx_v.at[k]])


def _read_acc(idx_v, rb, acc):
    for k in range(ZK):
        pltpu.sync_copy(acc.at[idx_v.at[k]], rb.at[k])


def _deg_body(dst_hbm, ridx_hbm, out_hbm, dst_v, idx_v, ones_v, zb, rb, acc,
              sem):
    sid = lax.axis_index("s")
    wid = _worker_id()
    pltpu.sync_copy(dst_hbm.at[wid], dst_v)
    pltpu.sync_copy(ridx_hbm.at[sid], idx_v)
    _fill_rows(ones_v, CHUNK, jnp.ones((L,), jnp.float32))
    _fill_rows(zb, ZC, jnp.zeros((L,), jnp.float32))
    _zero_acc(idx_v, zb, acc)
    plsc.subcore_barrier()

    def body(t, _):
        for b in range(DGRP):
            pltpu.async_copy(ones_v, acc.at[dst_v.at[t * DGRP + b]], sem,
                             add=True)
        for b in range(DGRP):
            pltpu.make_async_copy(ones_v, acc.at[dst_v.at[t * DGRP + b]],
                                  sem).wait()
        return 0

    lax.fori_loop(0, CPT // DGRP, body, 0)
    plsc.subcore_barrier()
    _read_acc(idx_v, rb, acc)
    pltpu.sync_copy(rb, out_hbm.at[wid])


_deg_kernel = pl.kernel(
    _deg_body,
    out_type=jax.ShapeDtypeStruct((NW, ZK, ZC, HID), jnp.float32),
    mesh=_MESH,
    scratch_types=[
        pltpu.VMEM((CPT, CHUNK), jnp.int32),
        pltpu.VMEM((ZK, ZC), jnp.int32),
        pltpu.VMEM((CHUNK, HID), jnp.float32),
        pltpu.VMEM((ZC, HID), jnp.float32),
        pltpu.VMEM((ZK, ZC, HID), jnp.float32),
        pltpu.VMEM_SHARED((N_PAD, HID), jnp.float32),
        pltpu.SemaphoreType.DMA,
    ],
    compiler_params=_SC_PARAMS,
)


def _msg_body(y_hbm, src_hbm, dst_hbm, ridx_hbm, out_hbm, src_v, dst_v,
              idx_v, rows, zb, rb, acc, ytab, *sems):
    sid = lax.axis_index("s")
    wid = _worker_id()
    pltpu.sync_copy(src_hbm.at[wid], src_v)
    pltpu.sync_copy(dst_hbm.at[wid], dst_v)
    pltpu.sync_copy(ridx_hbm.at[sid], idx_v)
    _fill_rows(zb, ZC, jnp.zeros((L,), jnp.float32))
    _zero_acc(idx_v, zb, acc)

    @pl.when(sid == 0)
    def _():
        pltpu.sync_copy(y_hbm, ytab)

    plsc.subcore_barrier()

    for b in range(NBUF):
        pltpu.async_copy(ytab.at[src_v.at[b]], rows.at[b], sems[b])

    def body(t, _):
        for b in range(NBUF):
            j = t * NBUF + b
            pltpu.make_async_copy(y_hbm.at[pl.ds(0, CHUNK)], rows.at[b],
                                  sems[b]).wait()
            pltpu.sync_copy(rows.at[b], acc.at[dst_v.at[j]], add=True)
            pltpu.async_copy(ytab.at[src_v.at[j + NBUF]], rows.at[b],
                             sems[b])
        return 0

    lax.fori_loop(0, CPT // NBUF - 1, body, 0)
    for b in range(NBUF):
        j = CPT - NBUF + b
        pltpu.make_async_copy(y_hbm.at[pl.ds(0, CHUNK)], rows.at[b],
                              sems[b]).wait()
        pltpu.sync_copy(rows.at[b], acc.at[dst_v.at[j]], add=True)
    plsc.subcore_barrier()
    _read_acc(idx_v, rb, acc)
    pltpu.sync_copy(rb, out_hbm.at[wid])


_msg_kernel = pl.kernel(
    _msg_body,
    out_type=jax.ShapeDtypeStruct((NW, ZK, ZC, HID), jnp.float32),
    mesh=_MESH,
    scratch_types=[
        pltpu.VMEM((CPT, CHUNK), jnp.int32),
        pltpu.VMEM((CPT, CHUNK), jnp.int32),
        pltpu.VMEM((ZK, ZC), jnp.int32),
        pltpu.VMEM((NBUF, CHUNK, HID), jnp.float32),
        pltpu.VMEM((ZC, HID), jnp.float32),
        pltpu.VMEM((ZK, ZC, HID), jnp.float32),
        pltpu.VMEM_SHARED((N_PAD, HID), jnp.float32),
        pltpu.VMEM_SHARED((N, HID), jnp.float32),
    ] + [pltpu.SemaphoreType.DMA] * NBUF,
    compiler_params=_SC_PARAMS,
)


def _edge_body(a_hbm, c_hbm, src_hbm, dst_hbm, out_hbm, a_v, c_v, src_v,
               dst_v, o_v):
    wid = _worker_id()
    pltpu.sync_copy(a_hbm, a_v)
    pltpu.sync_copy(c_hbm, c_v)
    pltpu.sync_copy(src_hbm.at[wid], src_v)
    pltpu.sync_copy(dst_hbm.at[wid], dst_v)

    def body(j, _):
        for k in range(CHUNK // L):
            sv = src_v[j, pl.ds(k * L, L)]
            dv = dst_v[j, pl.ds(k * L, L)]
            av = plsc.load_gather(a_v, [sv])
            cv = plsc.load_gather(c_v, [dv])
            o_v[pl.ds(j * CHUNK + k * L, L)] = av + cv
        return 0

    lax.fori_loop(0, CPT, body, 0)
    pltpu.sync_copy(o_v, out_hbm.at[pl.ds(wid * EPT, EPT)])


_edge_kernel = pl.kernel(
    _edge_body,
    out_type=jax.ShapeDtypeStruct((E_PAD,), jnp.float32),
    mesh=_MESH,
    scratch_types=[
        pltpu.VMEM((N_PAD,), jnp.float32),
        pltpu.VMEM((N_PAD,), jnp.float32),
        pltpu.VMEM((CPT, CHUNK), jnp.int32),
        pltpu.VMEM((CPT, CHUNK), jnp.int32),
        pltpu.VMEM((EPT,), jnp.float32),
    ],
    compiler_params=pltpu.CompilerParams(needs_layout_passes=False),
)


def _tc1_body(x_ref, w1_ref, degp_ref, y1_ref, dinv_ref):
    deg = (degp_ref[0:N, 0:1] + degp_ref[N_PAD:N_PAD + N, 0:1]) + 1.0
    dinv = lax.rsqrt(deg)
    y1 = dinv * jnp.dot(x_ref[...], w1_ref[...],
                        preferred_element_type=jnp.float32)
    y1_ref[...] = y1
    dinv_ref[...] = dinv


_tc1 = pl.pallas_call(
    _tc1_body,
    out_shape=(
        jax.ShapeDtypeStruct((N, HID), jnp.float32),
        jax.ShapeDtypeStruct((N, 1), jnp.float32),
    ),
)


def _tc2_body(sp_ref, y1_ref, dinv_ref, w2_ref, b1_ref, y2_ref):
    s1 = sp_ref[0:N, :] + sp_ref[N_PAD:N_PAD + N, :]
    dinv = dinv_ref[...]
    h1 = jnp.maximum(dinv * (s1 + y1_ref[...]) + b1_ref[...], 0.0)
    y2_ref[...] = dinv * jnp.dot(h1, w2_ref[...],
                                 preferred_element_type=jnp.float32)


_tc2 = pl.pallas_call(
    _tc2_body,
    out_shape=jax.ShapeDtypeStruct((N, HID), jnp.float32),
)


def _tc3_body(sp_ref, y2_ref, dinv_ref, wfc_ref, b2_ref, bfc_ref, a_ref,
              c_ref):
    s2 = sp_ref[0:N, :] + sp_ref[N_PAD:N_PAD + N, :]
    dinv = dinv_ref[...]
    h2 = jnp.maximum(dinv * (s2 + y2_ref[...]) + b2_ref[...], 0.0)
    a = jnp.dot(h2, wfc_ref[0:HID, :],
                preferred_element_type=jnp.float32) + bfc_ref[...]
    c = jnp.dot(h2, wfc_ref[HID:2 * HID, :],
                preferred_element_type=jnp.float32)
    a_ref[0:N, :] = a
    a_ref[N:N_PAD, :] = jnp.zeros((N_PAD - N, 1), jnp.float32)
    c_ref[0:N, :] = c
    c_ref[N:N_PAD, :] = jnp.zeros((N_PAD - N, 1), jnp.float32)


_tc3 = pl.pallas_call(
    _tc3_body,
    out_shape=(
        jax.ShapeDtypeStruct((N_PAD, 1), jnp.float32),
        jax.ShapeDtypeStruct((N_PAD, 1), jnp.float32),
    ),
)


@jax.jit
def kernel(x, edge_index, W1, b1, W2, b2, Wfc, bfc):
    src = edge_index[0].astype(jnp.int32)
    dst = edge_index[1].astype(jnp.int32)
    pad = E_PAD - E
    src_p = jnp.concatenate([src, jnp.zeros((pad,), jnp.int32)])
    dst_p = jnp.concatenate([dst, jnp.full((pad,), N, jnp.int32)])
    src_p = src_p.reshape(NW, CPT, CHUNK)
    dst_p = dst_p.reshape(NW, CPT, CHUNK)
    ridx = jnp.arange(N_PAD, dtype=jnp.int32).reshape(NS, ZK, ZC)

    degp = _deg_kernel(dst_p, ridx).reshape(NC * N_PAD, HID)
    y1, dinv = _tc1(x, W1, degp)
    sp1 = _msg_kernel(y1, src_p, dst_p, ridx).reshape(NC * N_PAD, HID)
    y2 = _tc2(sp1, y1, dinv, W2, b1.reshape(1, HID))
    sp2 = _msg_kernel(y2, src_p, dst_p, ridx).reshape(NC * N_PAD, HID)
    a_pad, c_pad = _tc3(sp2, y2, dinv, Wfc, b2.reshape(1, HID),
                        bfc.reshape(1, 1))
    pred = _edge_kernel(a_pad.reshape(N_PAD), c_pad.reshape(N_PAD), src_p,
                        dst_p)
    return pred[:E]
